# baseline (speedup 1.0000x reference)
import functools

import numpy as np
import jax
import jax.numpy as jnp
from jax import lax
from jax.experimental import pallas as pl
from jax.experimental.pallas import tpu as pltpu
from jax.experimental.pallas import tpu_sc as plsc

F32 = jnp.float32

B = 64
QS, CS = 40, 56
MS = 64
PAIR = QS + CS
N = B * PAIR
E = 49152
DIN = 64
D = 64
EENC = 32
TD = 64

NC, NS = 2, 16
NW = NC * NS
EPW = E // NW
CH = 128
NCH = EPW // CH
ROWS_PER_TILE = N // NS

RB = 512
NRB = N // RB

PP = 8
TG = B // PP

_G2 = np.arange(E // 2)
_EDGE_ORDER = np.empty((E,), np.int32)
_EDGE_ORDER[0::2] = (_G2 // 2048) * 4096 + _G2 % 2048
_EDGE_ORDER[1::2] = _EDGE_ORDER[0::2] + 2048


_EB = 4096


def _enc_edges_body(x_ref, we_ref, be_ref, wm_ref, bm_ref, o_ref):
    enc = jnp.dot(x_ref[...], we_ref[...], preferred_element_type=F32) + be_ref[...]
    ce = jnp.dot(enc, wm_ref[...], preferred_element_type=F32) + bm_ref[...]
    o_ref[...] = jnp.concatenate([ce[:_EB // 2], ce[_EB // 2:]], axis=1)


def _enc_edges(x, we, be, wm, bm):
    return pl.pallas_call(
        _enc_edges_body,
        grid=(E // _EB,),
        in_specs=[
            pl.BlockSpec((_EB, 16), lambda i: (i, 0)),
            pl.BlockSpec((16, EENC), lambda i: (0, 0)),
            pl.BlockSpec((1, EENC), lambda i: (0, 0)),
            pl.BlockSpec((EENC, D), lambda i: (0, 0)),
            pl.BlockSpec((1, D), lambda i: (0, 0)),
        ],
        out_specs=pl.BlockSpec((_EB // 2, 2 * D), lambda i: (i, 0)),
        out_shape=jax.ShapeDtypeStruct((E // 2, 2 * D), F32),
    )(x, we, be, wm, bm)


def _proj_ab(comb, wma_ref, wmb_ref):
    wmab = jnp.concatenate([wma_ref[...], wmb_ref[...]], axis=1)
    return jnp.dot(comb, wmab, preferred_element_type=F32)


def _layer1_body(x_ref, wen_ref, ben_ref, wc1a_ref, bc1_ref, wc2_ref, bc2_ref,
                 wma_ref, wmb_ref, comb_ref, ab_ref):
    h0 = jnp.dot(x_ref[...], wen_ref[...], preferred_element_type=F32) + ben_ref[...]
    y = jnp.maximum(
        jnp.dot(h0, wc1a_ref[...], preferred_element_type=F32) + bc1_ref[...],
        0.0)
    comb = jnp.dot(y, wc2_ref[...], preferred_element_type=F32) + bc2_ref[...]
    comb_ref[...] = comb
    ab_ref[...] = _proj_ab(comb, wma_ref, wmb_ref)


def _layer1(x, wen, ben, wc1a, bc1, wc2, bc2, wma, wmb):
    return pl.pallas_call(
        _layer1_body,
        grid=(NRB,),
        in_specs=[
            pl.BlockSpec((RB, DIN), lambda i: (i, 0)),
            pl.BlockSpec((DIN, D), lambda i: (0, 0)),
            pl.BlockSpec((1, D), lambda i: (0, 0)),
            pl.BlockSpec((D, 2 * D), lambda i: (0, 0)),
            pl.BlockSpec((1, 2 * D), lambda i: (0, 0)),
            pl.BlockSpec((2 * D, D), lambda i: (0, 0)),
            pl.BlockSpec((1, D), lambda i: (0, 0)),
            pl.BlockSpec((D, D), lambda i: (0, 0)),
            pl.BlockSpec((D, D), lambda i: (0, 0)),
        ],
        out_specs=[
            pl.BlockSpec((RB, D), lambda i: (i, 0)),
            pl.BlockSpec((RB, 2 * D), lambda i: (i, 0)),
        ],
        out_shape=[
            jax.ShapeDtypeStruct((N, D), F32),
            jax.ShapeDtypeStruct((N, 2 * D), F32),
        ],
    )(x, wen, ben, wc1a, bc1, wc2, bc2, wma, wmb)


def _unpair(app):
    pieces = []
    for g in range(app.shape[0] // 128):
        blk = app[128 * g:128 * (g + 1)]
        pieces += [blk[:, :D], blk[:, D:]]
    return jnp.concatenate(pieces, axis=0)


def _h_from(combp, gp_ref, wu1_ref, wu2_ref, bu_ref):
    agg = _unpair(gp_ref[0] + gp_ref[1])
    return jnp.maximum(
        jnp.dot(combp, wu1_ref[...], preferred_element_type=F32)
        + jnp.dot(agg, wu2_ref[...], preferred_element_type=F32)
        + bu_ref[...], 0.0)


def _layer_next_body(cp_ref, gp_ref, wu1_ref, wu2_ref, bu_ref, wc1a_ref,
                     bc1_ref, wc2_ref, bc2_ref, wma_ref, wmb_ref,
                     h_ref, comb_ref, ab_ref):
    h = _h_from(cp_ref[...], gp_ref, wu1_ref, wu2_ref, bu_ref)
    h_ref[...] = h
    y = jnp.maximum(
        jnp.dot(h, wc1a_ref[...], preferred_element_type=F32) + bc1_ref[...],
        0.0)
    comb = jnp.dot(y, wc2_ref[...], preferred_element_type=F32) + bc2_ref[...]
    comb_ref[...] = comb
    ab_ref[...] = _proj_ab(comb, wma_ref, wmb_ref)


def _layer_next_wi_body(cp_ref, gp_ref, int_ref, wu1_ref, wu2_ref, bu_ref,
                        wc1a_ref, wc1b_ref, bc1_ref, wc2_ref, bc2_ref,
                        wma_ref, wmb_ref, h_ref, comb_ref, ab_ref):
    h = _h_from(cp_ref[...], gp_ref, wu1_ref, wu2_ref, bu_ref)
    h_ref[...] = h
    y = jnp.maximum(
        jnp.dot(h, wc1a_ref[...], preferred_element_type=F32)
        + jnp.dot(int_ref[...], wc1b_ref[...], preferred_element_type=F32)
        + bc1_ref[...], 0.0)
    comb = jnp.dot(y, wc2_ref[...], preferred_element_type=F32) + bc2_ref[...]
    comb_ref[...] = comb
    ab_ref[...] = _proj_ab(comb, wma_ref, wmb_ref)


_ROW_SPEC = pl.BlockSpec((RB, D), lambda i: (i, 0))
_AB_SPEC = pl.BlockSpec((RB, 2 * D), lambda i: (i, 0))
_AGG_SPEC = pl.BlockSpec((NC, RB // 2, 2 * D), lambda i: (0, i, 0))
_W64_SPEC = pl.BlockSpec((D, D), lambda i: (0, 0))
_B64_SPEC = pl.BlockSpec((1, D), lambda i: (0, 0))
_W128_SPEC = pl.BlockSpec((D, 2 * D), lambda i: (0, 0))
_B128_SPEC = pl.BlockSpec((1, 2 * D), lambda i: (0, 0))
_W2I_SPEC = pl.BlockSpec((2 * D, D), lambda i: (0, 0))

_L3_OUT = [_ROW_SPEC, _ROW_SPEC, _AB_SPEC]
_L3_SHAPE = [jax.ShapeDtypeStruct((N, D), F32),
             jax.ShapeDtypeStruct((N, D), F32),
             jax.ShapeDtypeStruct((N, 2 * D), F32)]


def _layer_next(cp, gp2, wu1, wu2, bu, wc1a, bc1, wc2, bc2, wma, wmb):
    return pl.pallas_call(
        _layer_next_body,
        grid=(NRB,),
        in_specs=[_ROW_SPEC, _AGG_SPEC, _W64_SPEC, _W64_SPEC, _B64_SPEC,
                  _W128_SPEC, _B128_SPEC, _W2I_SPEC, _B64_SPEC, _W64_SPEC,
                  _W64_SPEC],
        out_specs=_L3_OUT,
        out_shape=_L3_SHAPE,
    )(cp, gp2, wu1, wu2, bu, wc1a, bc1, wc2, bc2, wma, wmb)


def _layer_next_wi(cp, gp2, inter, wu1, wu2, bu, wc1a, wc1b, bc1, wc2, bc2,
                   wma, wmb):
    return pl.pallas_call(
        _layer_next_wi_body,
        grid=(NRB,),
        in_specs=[_ROW_SPEC, _AGG_SPEC, _ROW_SPEC, _W64_SPEC, _W64_SPEC,
                  _B64_SPEC, _W128_SPEC, _W128_SPEC, _B128_SPEC, _W2I_SPEC,
                  _B64_SPEC, _W64_SPEC, _W64_SPEC],
        out_specs=_L3_OUT,
        out_shape=_L3_SHAPE,
    )(cp, gp2, inter, wu1, wu2, bu, wc1a, wc1b, bc1, wc2, bc2, wma, wmb)


def _pad_qc(h, w):
    zq = jnp.zeros((MS - QS, w), F32)
    zc = jnp.zeros((MS - CS, w), F32)
    qs, cs = [], []
    for p in range(PP):
        qs += [h[PAIR * p:PAIR * p + QS], zq]
        cs += [h[PAIR * p + QS:PAIR * (p + 1)], zc]
    return jnp.concatenate(qs, axis=0), jnp.concatenate(cs, axis=0)


def _masked_transform(h3, wt1_ref, bt1_ref, wt2_ref, bt2_ref):
    q3, c3 = _pad_qc(h3, D)

    def transform(x):
        y = jnp.maximum(
            jnp.dot(x, wt1_ref[...], preferred_element_type=F32) + bt1_ref[...],
            0.0)
        return jnp.dot(y, wt2_ref[...], preferred_element_type=F32) + bt2_ref[...]

    rid = lax.broadcasted_iota(jnp.int32, (PP * MS, 1), 0) % MS
    mq = jnp.where(rid < QS, transform(q3), 0.0)
    mc = jnp.where(rid < CS, transform(c3), 0.0)
    return mq, mc


def _plan_from(mq, mc):
    sims = []
    for p in range(PP):
        s = lax.dot_general(mq[MS * p:MS * (p + 1)], mc[MS * p:MS * (p + 1)],
                            (((1,), (1,)), ((), ())),
                            preferred_element_type=F32)
        sims.append(s.reshape(1, MS, MS))
    la = jnp.concatenate(sims, axis=0) * 10.0
    for _ in range(10):
        m = jnp.max(la, axis=2, keepdims=True)
        la = la - (m + jnp.log(jnp.sum(jnp.exp(la - m), axis=2, keepdims=True)))
        m = jnp.max(la, axis=1, keepdims=True)
        la = la - (m + jnp.log(jnp.sum(jnp.exp(la - m), axis=1, keepdims=True)))
    return jnp.exp(la)


def _tail0_body(h1_ref, h2_ref, c3_ref, g3_ref, wu1_ref, wu2_ref, bu_ref,
                wt1_ref, bt1_ref, wt2_ref, bt2_ref, s1_ref, s2_ref):
    h3 = _h_from(c3_ref[...], g3_ref, wu1_ref, wu2_ref, bu_ref)
    mq, mc = _masked_transform(h3, wt1_ref, bt1_ref, wt2_ref, bt2_ref)
    plan3 = _plan_from(mq, mc)
    h12 = jnp.concatenate([h1_ref[...], h2_ref[...]], axis=1)
    q12, c12 = _pad_qc(h12, 2 * D)
    s_pieces = []
    for p in range(PP):
        plan = plan3[p]
        cb = c12[MS * p:MS * (p + 1)]
        qb = q12[MS * p:MS * (p + 1)]
        outq = jnp.dot(plan, cb, preferred_element_type=F32)
        outc = lax.dot_general(plan, qb, (((0,), (0,)), ((), ())),
                               preferred_element_type=F32)
        s_pieces += [outq[:QS], outc[:CS]]
    s12 = jnp.concatenate(s_pieces, axis=0)
    s1_ref[...] = s12[:, :D]
    s2_ref[...] = s12[:, D:]


def _tail0(h1, h2, c3, g3, wu1, wu2, bu, wt1, bt1, wt2, bt2):
    blk = pl.BlockSpec((PP * PAIR, D), lambda i: (i, 0))
    gblk = pl.BlockSpec((NC, PP * PAIR // 2, 2 * D), lambda i: (0, i, 0))
    wt = pl.BlockSpec((TD, TD), lambda i: (0, 0))
    bt = pl.BlockSpec((1, TD), lambda i: (0, 0))
    return pl.pallas_call(
        _tail0_body,
        grid=(TG,),
        in_specs=[blk, blk, blk, gblk, wt, wt, bt, wt, bt, wt, bt],
        out_specs=[blk, blk],
        out_shape=[jax.ShapeDtypeStruct((N, D), F32),
                   jax.ShapeDtypeStruct((N, D), F32)],
    )(h1, h2, c3, g3, wu1, wu2, bu, wt1, bt1, wt2, bt2)


def _tail1_body(c3_ref, g3_ref, wu1_ref, wu2_ref, bu_ref, wt1_ref, bt1_ref,
                wt2_ref, bt2_ref, o_ref):
    h3 = _h_from(c3_ref[...], g3_ref, wu1_ref, wu2_ref, bu_ref)
    mq, mc = _masked_transform(h3, wt1_ref, bt1_ref, wt2_ref, bt2_ref)
    plan3 = _plan_from(mq, mc)
    rows = []
    for p in range(PP):
        mqb = mq[MS * p:MS * (p + 1)]
        mcb = mc[MS * p:MS * (p + 1)]
        r = mqb - jnp.dot(plan3[p], mcb, preferred_element_type=F32)
        s = -jnp.sqrt(jnp.sum(r * r) + 1e-12)
        rows.append(jnp.full((1, 128), s, F32))
    o_ref[...] = jnp.concatenate(rows, axis=0)


def _tail1(c3, g3, wu1, wu2, bu, wt1, bt1, wt2, bt2):
    blk = pl.BlockSpec((PP * PAIR, D), lambda i: (i, 0))
    gblk = pl.BlockSpec((NC, PP * PAIR // 2, 2 * D), lambda i: (0, i, 0))
    wt = pl.BlockSpec((TD, TD), lambda i: (0, 0))
    bt = pl.BlockSpec((1, TD), lambda i: (0, 0))
    return pl.pallas_call(
        _tail1_body,
        grid=(TG,),
        in_specs=[blk, gblk, wt, wt, bt, wt, bt, wt, bt],
        out_specs=pl.BlockSpec((PP, 128), lambda i: (i, 0)),
        out_shape=jax.ShapeDtypeStruct((B, 128), F32),
    )(c3, g3, wu1, wu2, bu, wt1, bt1, wt2, bt2)


_SC_MESH = plsc.VectorSubcoreMesh(core_axis_name="c", subcore_axis_name="s")


@functools.partial(
    pl.kernel,
    out_type=jax.ShapeDtypeStruct((NC, N // 2, 2 * D), F32),
    mesh=_SC_MESH,
    compiler_params=pltpu.CompilerParams(use_tc_tiling_on_sc=False),
    scratch_types=[
        pltpu.VMEM((NCH, CH), jnp.int32),
        pltpu.VMEM((NCH, CH), jnp.int32),
        pltpu.VMEM((NCH, CH), jnp.int32),
        pltpu.VMEM((CH, D), F32),
        pltpu.VMEM((CH, D), F32),
        pltpu.VMEM((CH // 2, 2 * D), F32),
        pltpu.VMEM_SHARED((N, D), F32),
        pltpu.SemaphoreType.DMA,
        pltpu.SemaphoreType.DMA,
        pltpu.SemaphoreType.DMA,
    ],
)
def _edge_sc(ab_hbm, ce_hbm, f2_hbm, t2_hbm, t_hbm, out_hbm,
             fidx2, tidx2, tidx, buf_a, buf_b, buf_c, agg,
             sem_a, sem_b, sem_c):
    cid = lax.axis_index("c")
    sid = lax.axis_index("s")
    wid = cid * NS + sid

    def zrow(r, carry):
        for q in range(D // 16):
            buf_a[r, pl.ds(q * 16, 16)] = jnp.zeros((16,), F32)
        return carry

    lax.fori_loop(0, CH, zrow, 0)
    for k in range(ROWS_PER_TILE // CH):
        pltpu.sync_copy(buf_a, agg.at[pl.ds(sid * ROWS_PER_TILE + k * CH, CH)])
    plsc.subcore_barrier()

    pltpu.sync_copy(f2_hbm.at[wid], fidx2)
    pltpu.sync_copy(t2_hbm.at[wid], tidx2)
    pltpu.sync_copy(t_hbm.at[wid], tidx)

    def chunk(j, carry):
        ca = pltpu.async_copy(ab_hbm.at[fidx2.at[j]], buf_a, sem_a)
        cb = pltpu.async_copy(ab_hbm.at[tidx2.at[j]], buf_b, sem_b)
        cc = pltpu.async_copy(
            ce_hbm.at[pl.ds((wid * NCH + j) * (CH // 2), CH // 2)], buf_c,
            sem_c)
        ca.wait()
        cb.wait()
        cc.wait()

        @plsc.parallel_loop(0, CH // 2, unroll=2)
        def _rows(rp):
            r0 = 2 * rp
            r1 = r0 + 1
            for q in range(D // 16):
                sl = pl.ds(q * 16, 16)
                v = buf_a[r0, sl] + buf_b[r0, sl] + buf_c[rp, sl]
                buf_a[r0, sl] = jnp.maximum(v, 0.0)
            for q in range(D // 16):
                sl = pl.ds(q * 16, 16)
                sl2 = pl.ds(D + q * 16, 16)
                v = buf_a[r1, sl] + buf_b[r1, sl] + buf_c[rp, sl2]
                buf_a[r1, sl] = jnp.maximum(v, 0.0)

        pltpu.sync_copy(buf_a, agg.at[tidx.at[j]], add=True)
        return carry

    lax.fori_loop(0, NCH, chunk, 0)
    plsc.subcore_barrier()

    for k in range(ROWS_PER_TILE // CH):
        m = sid * (ROWS_PER_TILE // CH) + k
        pltpu.sync_copy(
            agg.at[pl.ds(m * CH, CH)],
            out_hbm.at[cid, pl.ds((m // 2) * CH, CH), pl.ds((m % 2) * D, D)])


def kernel(node_features, edge_features, from_idx, to_idx, W_enc_n, b_enc_n,
           W_enc_e, b_enc_e, Wc1, bc1, Wc2, bc2, W_msg, b_msg, W_upd, b_upd,
           Wt1, bt1, Wt2, bt2):
    order = jnp.asarray(_EDGE_ORDER)
    fi = from_idx.astype(jnp.int32)[order].reshape(NW, NCH, CH)
    ti = to_idx.astype(jnp.int32)[order].reshape(NW, NCH, CH)
    fi2 = fi * 2
    ti2 = ti * 2 + 1
    wc1a, wc1b = Wc1[:D], Wc1[D:]
    wma, wmb, wmc = W_msg[:D], W_msg[D:2 * D], W_msg[2 * D:]
    wu1, wu2 = W_upd[:D], W_upd[D:]
    bc1r = bc1.reshape(1, 2 * D)
    bc2r = bc2.reshape(1, D)
    bur = b_upd.reshape(1, D)
    bt1r = bt1.reshape(1, TD)
    bt2r = bt2.reshape(1, TD)

    ce2 = _enc_edges(edge_features, W_enc_e, b_enc_e.reshape(1, EENC), wmc,
                     b_msg.reshape(1, D))

    def edge(ab):
        g = _edge_sc(ab.reshape(2 * N, D), ce2, fi2, ti2, ti)
        return g, g

    c1, ab1 = _layer1(node_features, W_enc_n, b_enc_n.reshape(1, D),
                      wc1a, bc1r, Wc2, bc2r, wma, wmb)
    g1, gp1 = edge(ab1)
    h1, c2, ab2 = _layer_next(c1, gp1, wu1, wu2, bur, wc1a, bc1r, Wc2,
                              bc2r, wma, wmb)
    _, gp2 = edge(ab2)
    h2, c3, ab3 = _layer_next(c2, gp2, wu1, wu2, bur, wc1a, bc1r, Wc2,
                              bc2r, wma, wmb)
    _, gp3 = edge(ab3)
    s1, s2 = _tail0(h1, h2, c3, gp3, wu1, wu2, bur, Wt1, bt1r, Wt2, bt2r)
    _, c4, ab4 = _layer_next_wi(c1, gp1, s1, wu1, wu2, bur, wc1a, wc1b,
                                bc1r, Wc2, bc2r, wma, wmb)
    _, gp4 = edge(ab4)
    _, c5, ab5 = _layer_next_wi(c4, gp4, s2, wu1, wu2, bur, wc1a, wc1b,
                                bc1r, Wc2, bc2r, wma, wmb)
    _, gp5 = edge(ab5)
    out = _tail1(c5, gp5, wu1, wu2, bur, Wt1, bt1r, Wt2, bt2r)
    return out[:, 0]


# speedup vs baseline: 1.1085x; 1.1085x over previous
import functools

import numpy as np
import jax
import jax.numpy as jnp
from jax import lax
from jax.experimental import pallas as pl
from jax.experimental.pallas import tpu as pltpu
from jax.experimental.pallas import tpu_sc as plsc

F32 = jnp.float32

B = 64
QS, CS = 40, 56
MS = 64
PAIR = QS + CS
N = B * PAIR
E = 49152
DIN = 64
D = 64
EENC = 32
TD = 64

NC, NS = 2, 16
NW = NC * NS
EPW = E // NW
CH = 128
NCH = EPW // CH
ROWS_PER_TILE = N // NS

RB = 1024
NRB = N // RB

PP = 16
TG = B // PP

_G2 = np.arange(E // 2)
_EDGE_ORDER = np.empty((E,), np.int32)
_EDGE_ORDER[0::2] = (_G2 // 2048) * 4096 + _G2 % 2048
_EDGE_ORDER[1::2] = _EDGE_ORDER[0::2] + 2048


_EB = 4096


def _enc_edges_body(x_ref, we_ref, be_ref, wm_ref, bm_ref, o_ref):
    enc = jnp.dot(x_ref[...], we_ref[...], preferred_element_type=F32) + be_ref[...]
    ce = jnp.dot(enc, wm_ref[...], preferred_element_type=F32) + bm_ref[...]
    o_ref[...] = jnp.concatenate([ce[:_EB // 2], ce[_EB // 2:]], axis=1)


def _enc_edges(x, we, be, wm, bm):
    return pl.pallas_call(
        _enc_edges_body,
        grid=(E // _EB,),
        in_specs=[
            pl.BlockSpec((_EB, 16), lambda i: (i, 0)),
            pl.BlockSpec((16, EENC), lambda i: (0, 0)),
            pl.BlockSpec((1, EENC), lambda i: (0, 0)),
            pl.BlockSpec((EENC, D), lambda i: (0, 0)),
            pl.BlockSpec((1, D), lambda i: (0, 0)),
        ],
        out_specs=pl.BlockSpec((_EB // 2, 2 * D), lambda i: (i, 0)),
        out_shape=jax.ShapeDtypeStruct((E // 2, 2 * D), F32),
    )(x, we, be, wm, bm)


def _proj_ab(comb, wma_ref, wmb_ref):
    wmab = jnp.concatenate([wma_ref[...], wmb_ref[...]], axis=1)
    return jnp.dot(comb, wmab, preferred_element_type=F32)


def _layer1_body(x_ref, wen_ref, ben_ref, wc1a_ref, bc1_ref, wc2_ref, bc2_ref,
                 wma_ref, wmb_ref, comb_ref, ab_ref):
    h0 = jnp.dot(x_ref[...], wen_ref[...], preferred_element_type=F32) + ben_ref[...]
    y = jnp.maximum(
        jnp.dot(h0, wc1a_ref[...], preferred_element_type=F32) + bc1_ref[...],
        0.0)
    comb = jnp.dot(y, wc2_ref[...], preferred_element_type=F32) + bc2_ref[...]
    comb_ref[...] = comb
    ab_ref[...] = _proj_ab(comb, wma_ref, wmb_ref)


def _layer1(x, wen, ben, wc1a, bc1, wc2, bc2, wma, wmb):
    return pl.pallas_call(
        _layer1_body,
        grid=(NRB,),
        in_specs=[
            pl.BlockSpec((RB, DIN), lambda i: (i, 0)),
            pl.BlockSpec((DIN, D), lambda i: (0, 0)),
            pl.BlockSpec((1, D), lambda i: (0, 0)),
            pl.BlockSpec((D, 2 * D), lambda i: (0, 0)),
            pl.BlockSpec((1, 2 * D), lambda i: (0, 0)),
            pl.BlockSpec((2 * D, D), lambda i: (0, 0)),
            pl.BlockSpec((1, D), lambda i: (0, 0)),
            pl.BlockSpec((D, D), lambda i: (0, 0)),
            pl.BlockSpec((D, D), lambda i: (0, 0)),
        ],
        out_specs=[
            pl.BlockSpec((RB, D), lambda i: (i, 0)),
            pl.BlockSpec((RB, 2 * D), lambda i: (i, 0)),
        ],
        out_shape=[
            jax.ShapeDtypeStruct((N, D), F32),
            jax.ShapeDtypeStruct((N, 2 * D), F32),
        ],
    )(x, wen, ben, wc1a, bc1, wc2, bc2, wma, wmb)


def _unpair(app):
    pieces = []
    for g in range(app.shape[0] // 128):
        blk = app[128 * g:128 * (g + 1)]
        pieces += [blk[:, :D], blk[:, D:]]
    return jnp.concatenate(pieces, axis=0)


def _h_from(combp, gp_ref, wu1_ref, wu2_ref, bu_ref):
    agg = _unpair(gp_ref[0] + gp_ref[1])
    return jnp.maximum(
        jnp.dot(combp, wu1_ref[...], preferred_element_type=F32)
        + jnp.dot(agg, wu2_ref[...], preferred_element_type=F32)
        + bu_ref[...], 0.0)


def _layer_next_body(cp_ref, gp_ref, wu1_ref, wu2_ref, bu_ref, wc1a_ref,
                     bc1_ref, wc2_ref, bc2_ref, wma_ref, wmb_ref,
                     h_ref, comb_ref, ab_ref):
    h = _h_from(cp_ref[...], gp_ref, wu1_ref, wu2_ref, bu_ref)
    h_ref[...] = h
    y = jnp.maximum(
        jnp.dot(h, wc1a_ref[...], preferred_element_type=F32) + bc1_ref[...],
        0.0)
    comb = jnp.dot(y, wc2_ref[...], preferred_element_type=F32) + bc2_ref[...]
    comb_ref[...] = comb
    ab_ref[...] = _proj_ab(comb, wma_ref, wmb_ref)


def _layer_next_wi_body(cp_ref, gp_ref, int_ref, wu1_ref, wu2_ref, bu_ref,
                        wc1a_ref, wc1b_ref, bc1_ref, wc2_ref, bc2_ref,
                        wma_ref, wmb_ref, h_ref, comb_ref, ab_ref):
    h = _h_from(cp_ref[...], gp_ref, wu1_ref, wu2_ref, bu_ref)
    h_ref[...] = h
    y = jnp.maximum(
        jnp.dot(h, wc1a_ref[...], preferred_element_type=F32)
        + jnp.dot(int_ref[...], wc1b_ref[...], preferred_element_type=F32)
        + bc1_ref[...], 0.0)
    comb = jnp.dot(y, wc2_ref[...], preferred_element_type=F32) + bc2_ref[...]
    comb_ref[...] = comb
    ab_ref[...] = _proj_ab(comb, wma_ref, wmb_ref)


_ROW_SPEC = pl.BlockSpec((RB, D), lambda i: (i, 0))
_AB_SPEC = pl.BlockSpec((RB, 2 * D), lambda i: (i, 0))
_AGG_SPEC = pl.BlockSpec((NC, RB // 2, 2 * D), lambda i: (0, i, 0))
_W64_SPEC = pl.BlockSpec((D, D), lambda i: (0, 0))
_B64_SPEC = pl.BlockSpec((1, D), lambda i: (0, 0))
_W128_SPEC = pl.BlockSpec((D, 2 * D), lambda i: (0, 0))
_B128_SPEC = pl.BlockSpec((1, 2 * D), lambda i: (0, 0))
_W2I_SPEC = pl.BlockSpec((2 * D, D), lambda i: (0, 0))

_L3_OUT = [_ROW_SPEC, _ROW_SPEC, _AB_SPEC]
_L3_SHAPE = [jax.ShapeDtypeStruct((N, D), F32),
             jax.ShapeDtypeStruct((N, D), F32),
             jax.ShapeDtypeStruct((N, 2 * D), F32)]


def _layer_next(cp, gp2, wu1, wu2, bu, wc1a, bc1, wc2, bc2, wma, wmb):
    return pl.pallas_call(
        _layer_next_body,
        grid=(NRB,),
        in_specs=[_ROW_SPEC, _AGG_SPEC, _W64_SPEC, _W64_SPEC, _B64_SPEC,
                  _W128_SPEC, _B128_SPEC, _W2I_SPEC, _B64_SPEC, _W64_SPEC,
                  _W64_SPEC],
        out_specs=_L3_OUT,
        out_shape=_L3_SHAPE,
    )(cp, gp2, wu1, wu2, bu, wc1a, bc1, wc2, bc2, wma, wmb)


def _layer_next_wi(cp, gp2, inter, wu1, wu2, bu, wc1a, wc1b, bc1, wc2, bc2,
                   wma, wmb):
    return pl.pallas_call(
        _layer_next_wi_body,
        grid=(NRB,),
        in_specs=[_ROW_SPEC, _AGG_SPEC, _ROW_SPEC, _W64_SPEC, _W64_SPEC,
                  _B64_SPEC, _W128_SPEC, _W128_SPEC, _B128_SPEC, _W2I_SPEC,
                  _B64_SPEC, _W64_SPEC, _W64_SPEC],
        out_specs=_L3_OUT,
        out_shape=_L3_SHAPE,
    )(cp, gp2, inter, wu1, wu2, bu, wc1a, wc1b, bc1, wc2, bc2, wma, wmb)


def _pad_qc(h, w):
    zq = jnp.zeros((MS - QS, w), F32)
    zc = jnp.zeros((MS - CS, w), F32)
    qs, cs = [], []
    for p in range(PP):
        qs += [h[PAIR * p:PAIR * p + QS], zq]
        cs += [h[PAIR * p + QS:PAIR * (p + 1)], zc]
    return jnp.concatenate(qs, axis=0), jnp.concatenate(cs, axis=0)


def _masked_transform(h3, wt1_ref, bt1_ref, wt2_ref, bt2_ref):
    q3, c3 = _pad_qc(h3, D)

    def transform(x):
        y = jnp.maximum(
            jnp.dot(x, wt1_ref[...], preferred_element_type=F32) + bt1_ref[...],
            0.0)
        return jnp.dot(y, wt2_ref[...], preferred_element_type=F32) + bt2_ref[...]

    rid = lax.broadcasted_iota(jnp.int32, (PP * MS, 1), 0) % MS
    mq = jnp.where(rid < QS, transform(q3), 0.0)
    mc = jnp.where(rid < CS, transform(c3), 0.0)
    return mq, mc


def _plan_from(mq, mc):
    sims = []
    for p in range(PP):
        s = lax.dot_general(mq[MS * p:MS * (p + 1)], mc[MS * p:MS * (p + 1)],
                            (((1,), (1,)), ((), ())),
                            preferred_element_type=F32)
        sims.append(s.reshape(1, MS, MS))
    la = jnp.concatenate(sims, axis=0) * 10.0
    for _ in range(10):
        m = jnp.max(la, axis=2, keepdims=True)
        la = la - (m + jnp.log(jnp.sum(jnp.exp(la - m), axis=2, keepdims=True)))
        m = jnp.max(la, axis=1, keepdims=True)
        la = la - (m + jnp.log(jnp.sum(jnp.exp(la - m), axis=1, keepdims=True)))
    return jnp.exp(la)


def _tail0_body(h1_ref, h2_ref, c3_ref, g3_ref, wu1_ref, wu2_ref, bu_ref,
                wt1_ref, bt1_ref, wt2_ref, bt2_ref, s1_ref, s2_ref):
    h3 = _h_from(c3_ref[...], g3_ref, wu1_ref, wu2_ref, bu_ref)
    mq, mc = _masked_transform(h3, wt1_ref, bt1_ref, wt2_ref, bt2_ref)
    plan3 = _plan_from(mq, mc)
    h12 = jnp.concatenate([h1_ref[...], h2_ref[...]], axis=1)
    q12, c12 = _pad_qc(h12, 2 * D)
    s_pieces = []
    for p in range(PP):
        plan = plan3[p]
        cb = c12[MS * p:MS * (p + 1)]
        qb = q12[MS * p:MS * (p + 1)]
        outq = jnp.dot(plan, cb, preferred_element_type=F32)
        outc = lax.dot_general(plan, qb, (((0,), (0,)), ((), ())),
                               preferred_element_type=F32)
        s_pieces += [outq[:QS], outc[:CS]]
    s12 = jnp.concatenate(s_pieces, axis=0)
    s1_ref[...] = s12[:, :D]
    s2_ref[...] = s12[:, D:]


def _tail0(h1, h2, c3, g3, wu1, wu2, bu, wt1, bt1, wt2, bt2):
    blk = pl.BlockSpec((PP * PAIR, D), lambda i: (i, 0))
    gblk = pl.BlockSpec((NC, PP * PAIR // 2, 2 * D), lambda i: (0, i, 0))
    wt = pl.BlockSpec((TD, TD), lambda i: (0, 0))
    bt = pl.BlockSpec((1, TD), lambda i: (0, 0))
    return pl.pallas_call(
        _tail0_body,
        grid=(TG,),
        in_specs=[blk, blk, blk, gblk, wt, wt, bt, wt, bt, wt, bt],
        out_specs=[blk, blk],
        out_shape=[jax.ShapeDtypeStruct((N, D), F32),
                   jax.ShapeDtypeStruct((N, D), F32)],
    )(h1, h2, c3, g3, wu1, wu2, bu, wt1, bt1, wt2, bt2)


def _tail1_body(c3_ref, g3_ref, wu1_ref, wu2_ref, bu_ref, wt1_ref, bt1_ref,
                wt2_ref, bt2_ref, o_ref):
    h3 = _h_from(c3_ref[...], g3_ref, wu1_ref, wu2_ref, bu_ref)
    mq, mc = _masked_transform(h3, wt1_ref, bt1_ref, wt2_ref, bt2_ref)
    plan3 = _plan_from(mq, mc)
    rows = []
    for p in range(PP):
        mqb = mq[MS * p:MS * (p + 1)]
        mcb = mc[MS * p:MS * (p + 1)]
        r = mqb - jnp.dot(plan3[p], mcb, preferred_element_type=F32)
        s = -jnp.sqrt(jnp.sum(r * r) + 1e-12)
        rows.append(jnp.full((1, 128), s, F32))
    o_ref[...] = jnp.concatenate(rows, axis=0)


def _tail1(c3, g3, wu1, wu2, bu, wt1, bt1, wt2, bt2):
    blk = pl.BlockSpec((PP * PAIR, D), lambda i: (i, 0))
    gblk = pl.BlockSpec((NC, PP * PAIR // 2, 2 * D), lambda i: (0, i, 0))
    wt = pl.BlockSpec((TD, TD), lambda i: (0, 0))
    bt = pl.BlockSpec((1, TD), lambda i: (0, 0))
    return pl.pallas_call(
        _tail1_body,
        grid=(TG,),
        in_specs=[blk, gblk, wt, wt, bt, wt, bt, wt, bt],
        out_specs=pl.BlockSpec((PP, 128), lambda i: (i, 0)),
        out_shape=jax.ShapeDtypeStruct((B, 128), F32),
    )(c3, g3, wu1, wu2, bu, wt1, bt1, wt2, bt2)


_SC_MESH = plsc.VectorSubcoreMesh(core_axis_name="c", subcore_axis_name="s")


@functools.partial(
    pl.kernel,
    out_type=jax.ShapeDtypeStruct((NC, N // 2, 2 * D), F32),
    mesh=_SC_MESH,
    compiler_params=pltpu.CompilerParams(use_tc_tiling_on_sc=False),
    scratch_types=[
        pltpu.VMEM((NCH, CH), jnp.int32),
        pltpu.VMEM((NCH, CH), jnp.int32),
        pltpu.VMEM((NCH, CH), jnp.int32),
        pltpu.VMEM((CH, D), F32),
        pltpu.VMEM((CH, D), F32),
        pltpu.VMEM((CH // 2, 2 * D), F32),
        pltpu.VMEM_SHARED((N, D), F32),
        pltpu.SemaphoreType.DMA,
        pltpu.SemaphoreType.DMA,
        pltpu.SemaphoreType.DMA,
    ],
)
def _edge_sc(ab_hbm, ce_hbm, f2_hbm, t2_hbm, t_hbm, out_hbm,
             fidx2, tidx2, tidx, buf_a, buf_b, buf_c, agg,
             sem_a, sem_b, sem_c):
    cid = lax.axis_index("c")
    sid = lax.axis_index("s")
    wid = cid * NS + sid

    def zrow(r, carry):
        for q in range(D // 16):
            buf_a[r, pl.ds(q * 16, 16)] = jnp.zeros((16,), F32)
        return carry

    lax.fori_loop(0, CH, zrow, 0)
    for k in range(ROWS_PER_TILE // CH):
        pltpu.sync_copy(buf_a, agg.at[pl.ds(sid * ROWS_PER_TILE + k * CH, CH)])
    plsc.subcore_barrier()

    pltpu.sync_copy(f2_hbm.at[wid], fidx2)
    pltpu.sync_copy(t2_hbm.at[wid], tidx2)
    pltpu.sync_copy(t_hbm.at[wid], tidx)

    def chunk(j, carry):
        ca = pltpu.async_copy(ab_hbm.at[fidx2.at[j]], buf_a, sem_a)
        cb = pltpu.async_copy(ab_hbm.at[tidx2.at[j]], buf_b, sem_b)
        cc = pltpu.async_copy(
            ce_hbm.at[pl.ds((wid * NCH + j) * (CH // 2), CH // 2)], buf_c,
            sem_c)
        ca.wait()
        cb.wait()
        cc.wait()

        def row(rp, inner):
            r0 = 2 * rp
            r1 = r0 + 1
            for q in range(D // 16):
                sl = pl.ds(q * 16, 16)
                v = buf_a[r0, sl] + buf_b[r0, sl] + buf_c[rp, sl]
                buf_a[r0, sl] = jnp.maximum(v, 0.0)
            for q in range(D // 16):
                sl = pl.ds(q * 16, 16)
                sl2 = pl.ds(D + q * 16, 16)
                v = buf_a[r1, sl] + buf_b[r1, sl] + buf_c[rp, sl2]
                buf_a[r1, sl] = jnp.maximum(v, 0.0)
            return inner

        lax.fori_loop(0, CH // 2, row, 0)
        pltpu.sync_copy(buf_a, agg.at[tidx.at[j]], add=True)
        return carry

    lax.fori_loop(0, NCH, chunk, 0)
    plsc.subcore_barrier()

    for k in range(ROWS_PER_TILE // CH):
        m = sid * (ROWS_PER_TILE // CH) + k
        pltpu.sync_copy(
            agg.at[pl.ds(m * CH, CH)],
            out_hbm.at[cid, pl.ds((m // 2) * CH, CH), pl.ds((m % 2) * D, D)])


def kernel(node_features, edge_features, from_idx, to_idx, W_enc_n, b_enc_n,
           W_enc_e, b_enc_e, Wc1, bc1, Wc2, bc2, W_msg, b_msg, W_upd, b_upd,
           Wt1, bt1, Wt2, bt2):
    order = jnp.asarray(_EDGE_ORDER)
    fi = from_idx.astype(jnp.int32)[order].reshape(NW, NCH, CH)
    ti = to_idx.astype(jnp.int32)[order].reshape(NW, NCH, CH)
    fi2 = fi * 2
    ti2 = ti * 2 + 1
    wc1a, wc1b = Wc1[:D], Wc1[D:]
    wma, wmb, wmc = W_msg[:D], W_msg[D:2 * D], W_msg[2 * D:]
    wu1, wu2 = W_upd[:D], W_upd[D:]
    bc1r = bc1.reshape(1, 2 * D)
    bc2r = bc2.reshape(1, D)
    bur = b_upd.reshape(1, D)
    bt1r = bt1.reshape(1, TD)
    bt2r = bt2.reshape(1, TD)

    ce2 = _enc_edges(edge_features, W_enc_e, b_enc_e.reshape(1, EENC), wmc,
                     b_msg.reshape(1, D))

    def edge(ab):
        g = _edge_sc(ab.reshape(2 * N, D), ce2, fi2, ti2, ti)
        return g, g

    c1, ab1 = _layer1(node_features, W_enc_n, b_enc_n.reshape(1, D),
                      wc1a, bc1r, Wc2, bc2r, wma, wmb)
    g1, gp1 = edge(ab1)
    h1, c2, ab2 = _layer_next(c1, gp1, wu1, wu2, bur, wc1a, bc1r, Wc2,
                              bc2r, wma, wmb)
    _, gp2 = edge(ab2)
    h2, c3, ab3 = _layer_next(c2, gp2, wu1, wu2, bur, wc1a, bc1r, Wc2,
                              bc2r, wma, wmb)
    _, gp3 = edge(ab3)
    s1, s2 = _tail0(h1, h2, c3, gp3, wu1, wu2, bur, Wt1, bt1r, Wt2, bt2r)
    _, c4, ab4 = _layer_next_wi(c1, gp1, s1, wu1, wu2, bur, wc1a, wc1b,
                                bc1r, Wc2, bc2r, wma, wmb)
    _, gp4 = edge(ab4)
    _, c5, ab5 = _layer_next_wi(c4, gp4, s2, wu1, wu2, bur, wc1a, wc1b,
                                bc1r, Wc2, bc2r, wma, wmb)
    _, gp5 = edge(ab5)
    out = _tail1(c5, gp5, wu1, wu2, bur, Wt1, bt1r, Wt2, bt2r)
    return out[:, 0]


# speedup vs baseline: 1.1172x; 1.0078x over previous
import functools

import numpy as np
import jax
import jax.numpy as jnp
from jax import lax
from jax.experimental import pallas as pl
from jax.experimental.pallas import tpu as pltpu
from jax.experimental.pallas import tpu_sc as plsc

F32 = jnp.float32

B = 64
QS, CS = 40, 56
MS = 64
PAIR = QS + CS
N = B * PAIR
E = 49152
DIN = 64
D = 64
EENC = 32
TD = 64

NC, NS = 2, 16
NW = NC * NS
EPW = E // NW
CH = 128
NCH = EPW // CH
ROWS_PER_TILE = N // NS

RB = 2048
NRB = N // RB

PP = 16
TG = B // PP

_G2 = np.arange(E // 2)
_EDGE_ORDER = np.empty((E,), np.int32)
_EDGE_ORDER[0::2] = (_G2 // 2048) * 4096 + _G2 % 2048
_EDGE_ORDER[1::2] = _EDGE_ORDER[0::2] + 2048


_EB = 4096


def _enc_edges_body(x_ref, we_ref, be_ref, wm_ref, bm_ref, o_ref):
    enc = jnp.dot(x_ref[...], we_ref[...], preferred_element_type=F32) + be_ref[...]
    ce = jnp.dot(enc, wm_ref[...], preferred_element_type=F32) + bm_ref[...]
    o_ref[...] = jnp.concatenate([ce[:_EB // 2], ce[_EB // 2:]], axis=1)


def _enc_edges(x, we, be, wm, bm):
    return pl.pallas_call(
        _enc_edges_body,
        grid=(E // _EB,),
        in_specs=[
            pl.BlockSpec((_EB, 16), lambda i: (i, 0)),
            pl.BlockSpec((16, EENC), lambda i: (0, 0)),
            pl.BlockSpec((1, EENC), lambda i: (0, 0)),
            pl.BlockSpec((EENC, D), lambda i: (0, 0)),
            pl.BlockSpec((1, D), lambda i: (0, 0)),
        ],
        out_specs=pl.BlockSpec((_EB // 2, 2 * D), lambda i: (i, 0)),
        out_shape=jax.ShapeDtypeStruct((E // 2, 2 * D), F32),
    )(x, we, be, wm, bm)


def _proj_ab(comb, wma_ref, wmb_ref):
    wmab = jnp.concatenate([wma_ref[...], wmb_ref[...]], axis=1)
    return jnp.dot(comb, wmab, preferred_element_type=F32)


def _layer1_body(x_ref, wen_ref, ben_ref, wc1a_ref, bc1_ref, wc2_ref, bc2_ref,
                 wma_ref, wmb_ref, comb_ref, ab_ref):
    h0 = jnp.dot(x_ref[...], wen_ref[...], preferred_element_type=F32) + ben_ref[...]
    y = jnp.maximum(
        jnp.dot(h0, wc1a_ref[...], preferred_element_type=F32) + bc1_ref[...],
        0.0)
    comb = jnp.dot(y, wc2_ref[...], preferred_element_type=F32) + bc2_ref[...]
    comb_ref[...] = comb
    ab_ref[...] = _proj_ab(comb, wma_ref, wmb_ref)


def _layer1(x, wen, ben, wc1a, bc1, wc2, bc2, wma, wmb):
    return pl.pallas_call(
        _layer1_body,
        grid=(NRB,),
        in_specs=[
            pl.BlockSpec((RB, DIN), lambda i: (i, 0)),
            pl.BlockSpec((DIN, D), lambda i: (0, 0)),
            pl.BlockSpec((1, D), lambda i: (0, 0)),
            pl.BlockSpec((D, 2 * D), lambda i: (0, 0)),
            pl.BlockSpec((1, 2 * D), lambda i: (0, 0)),
            pl.BlockSpec((2 * D, D), lambda i: (0, 0)),
            pl.BlockSpec((1, D), lambda i: (0, 0)),
            pl.BlockSpec((D, D), lambda i: (0, 0)),
            pl.BlockSpec((D, D), lambda i: (0, 0)),
        ],
        out_specs=[
            pl.BlockSpec((RB, D), lambda i: (i, 0)),
            pl.BlockSpec((RB, 2 * D), lambda i: (i, 0)),
        ],
        out_shape=[
            jax.ShapeDtypeStruct((N, D), F32),
            jax.ShapeDtypeStruct((N, 2 * D), F32),
        ],
    )(x, wen, ben, wc1a, bc1, wc2, bc2, wma, wmb)


def _unpair(app):
    pieces = []
    for g in range(app.shape[0] // 128):
        blk = app[128 * g:128 * (g + 1)]
        pieces += [blk[:, :D], blk[:, D:]]
    return jnp.concatenate(pieces, axis=0)


def _h_from(combp, gp_ref, wu1_ref, wu2_ref, bu_ref):
    agg = _unpair(gp_ref[0] + gp_ref[1])
    return jnp.maximum(
        jnp.dot(combp, wu1_ref[...], preferred_element_type=F32)
        + jnp.dot(agg, wu2_ref[...], preferred_element_type=F32)
        + bu_ref[...], 0.0)


def _layer_next_body(cp_ref, gp_ref, wu1_ref, wu2_ref, bu_ref, wc1a_ref,
                     bc1_ref, wc2_ref, bc2_ref, wma_ref, wmb_ref,
                     h_ref, comb_ref, ab_ref):
    h = _h_from(cp_ref[...], gp_ref, wu1_ref, wu2_ref, bu_ref)
    h_ref[...] = h
    y = jnp.maximum(
        jnp.dot(h, wc1a_ref[...], preferred_element_type=F32) + bc1_ref[...],
        0.0)
    comb = jnp.dot(y, wc2_ref[...], preferred_element_type=F32) + bc2_ref[...]
    comb_ref[...] = comb
    ab_ref[...] = _proj_ab(comb, wma_ref, wmb_ref)


def _layer_next_wi_body(cp_ref, gp_ref, int_ref, wu1_ref, wu2_ref, bu_ref,
                        wc1a_ref, wc1b_ref, bc1_ref, wc2_ref, bc2_ref,
                        wma_ref, wmb_ref, h_ref, comb_ref, ab_ref):
    h = _h_from(cp_ref[...], gp_ref, wu1_ref, wu2_ref, bu_ref)
    h_ref[...] = h
    y = jnp.maximum(
        jnp.dot(h, wc1a_ref[...], preferred_element_type=F32)
        + jnp.dot(int_ref[...], wc1b_ref[...], preferred_element_type=F32)
        + bc1_ref[...], 0.0)
    comb = jnp.dot(y, wc2_ref[...], preferred_element_type=F32) + bc2_ref[...]
    comb_ref[...] = comb
    ab_ref[...] = _proj_ab(comb, wma_ref, wmb_ref)


_ROW_SPEC = pl.BlockSpec((RB, D), lambda i: (i, 0))
_AB_SPEC = pl.BlockSpec((RB, 2 * D), lambda i: (i, 0))
_AGG_SPEC = pl.BlockSpec((NC, RB // 2, 2 * D), lambda i: (0, i, 0))
_W64_SPEC = pl.BlockSpec((D, D), lambda i: (0, 0))
_B64_SPEC = pl.BlockSpec((1, D), lambda i: (0, 0))
_W128_SPEC = pl.BlockSpec((D, 2 * D), lambda i: (0, 0))
_B128_SPEC = pl.BlockSpec((1, 2 * D), lambda i: (0, 0))
_W2I_SPEC = pl.BlockSpec((2 * D, D), lambda i: (0, 0))

_L3_OUT = [_ROW_SPEC, _ROW_SPEC, _AB_SPEC]
_L3_SHAPE = [jax.ShapeDtypeStruct((N, D), F32),
             jax.ShapeDtypeStruct((N, D), F32),
             jax.ShapeDtypeStruct((N, 2 * D), F32)]


def _layer_next(cp, gp2, wu1, wu2, bu, wc1a, bc1, wc2, bc2, wma, wmb):
    return pl.pallas_call(
        _layer_next_body,
        grid=(NRB,),
        in_specs=[_ROW_SPEC, _AGG_SPEC, _W64_SPEC, _W64_SPEC, _B64_SPEC,
                  _W128_SPEC, _B128_SPEC, _W2I_SPEC, _B64_SPEC, _W64_SPEC,
                  _W64_SPEC],
        out_specs=_L3_OUT,
        out_shape=_L3_SHAPE,
    )(cp, gp2, wu1, wu2, bu, wc1a, bc1, wc2, bc2, wma, wmb)


def _layer_next_wi(cp, gp2, inter, wu1, wu2, bu, wc1a, wc1b, bc1, wc2, bc2,
                   wma, wmb):
    return pl.pallas_call(
        _layer_next_wi_body,
        grid=(NRB,),
        in_specs=[_ROW_SPEC, _AGG_SPEC, _ROW_SPEC, _W64_SPEC, _W64_SPEC,
                  _B64_SPEC, _W128_SPEC, _W128_SPEC, _B128_SPEC, _W2I_SPEC,
                  _B64_SPEC, _W64_SPEC, _W64_SPEC],
        out_specs=_L3_OUT,
        out_shape=_L3_SHAPE,
    )(cp, gp2, inter, wu1, wu2, bu, wc1a, wc1b, bc1, wc2, bc2, wma, wmb)


def _pad_qc(h, w):
    zq = jnp.zeros((MS - QS, w), F32)
    zc = jnp.zeros((MS - CS, w), F32)
    qs, cs = [], []
    for p in range(PP):
        qs += [h[PAIR * p:PAIR * p + QS], zq]
        cs += [h[PAIR * p + QS:PAIR * (p + 1)], zc]
    return jnp.concatenate(qs, axis=0), jnp.concatenate(cs, axis=0)


def _masked_transform(h3, wt1_ref, bt1_ref, wt2_ref, bt2_ref):
    q3, c3 = _pad_qc(h3, D)

    def transform(x):
        y = jnp.maximum(
            jnp.dot(x, wt1_ref[...], preferred_element_type=F32) + bt1_ref[...],
            0.0)
        return jnp.dot(y, wt2_ref[...], preferred_element_type=F32) + bt2_ref[...]

    rid = lax.broadcasted_iota(jnp.int32, (PP * MS, 1), 0) % MS
    mq = jnp.where(rid < QS, transform(q3), 0.0)
    mc = jnp.where(rid < CS, transform(c3), 0.0)
    return mq, mc


def _plan_from(mq, mc):
    sims = []
    for p in range(PP):
        s = lax.dot_general(mq[MS * p:MS * (p + 1)], mc[MS * p:MS * (p + 1)],
                            (((1,), (1,)), ((), ())),
                            preferred_element_type=F32)
        sims.append(s.reshape(1, MS, MS))
    la = jnp.concatenate(sims, axis=0) * 10.0
    for _ in range(10):
        m = jnp.max(la, axis=2, keepdims=True)
        la = la - (m + jnp.log(jnp.sum(jnp.exp(la - m), axis=2, keepdims=True)))
        m = jnp.max(la, axis=1, keepdims=True)
        la = la - (m + jnp.log(jnp.sum(jnp.exp(la - m), axis=1, keepdims=True)))
    return jnp.exp(la)


def _tail0_body(h1_ref, h2_ref, c3_ref, g3_ref, wu1_ref, wu2_ref, bu_ref,
                wt1_ref, bt1_ref, wt2_ref, bt2_ref, s1_ref, s2_ref):
    h3 = _h_from(c3_ref[...], g3_ref, wu1_ref, wu2_ref, bu_ref)
    mq, mc = _masked_transform(h3, wt1_ref, bt1_ref, wt2_ref, bt2_ref)
    plan3 = _plan_from(mq, mc)
    h12 = jnp.concatenate([h1_ref[...], h2_ref[...]], axis=1)
    q12, c12 = _pad_qc(h12, 2 * D)
    s_pieces = []
    for p in range(PP):
        plan = plan3[p]
        cb = c12[MS * p:MS * (p + 1)]
        qb = q12[MS * p:MS * (p + 1)]
        outq = jnp.dot(plan, cb, preferred_element_type=F32)
        outc = lax.dot_general(plan, qb, (((0,), (0,)), ((), ())),
                               preferred_element_type=F32)
        s_pieces += [outq[:QS], outc[:CS]]
    s12 = jnp.concatenate(s_pieces, axis=0)
    s1_ref[...] = s12[:, :D]
    s2_ref[...] = s12[:, D:]


def _tail0(h1, h2, c3, g3, wu1, wu2, bu, wt1, bt1, wt2, bt2):
    blk = pl.BlockSpec((PP * PAIR, D), lambda i: (i, 0))
    gblk = pl.BlockSpec((NC, PP * PAIR // 2, 2 * D), lambda i: (0, i, 0))
    wt = pl.BlockSpec((TD, TD), lambda i: (0, 0))
    bt = pl.BlockSpec((1, TD), lambda i: (0, 0))
    return pl.pallas_call(
        _tail0_body,
        grid=(TG,),
        in_specs=[blk, blk, blk, gblk, wt, wt, bt, wt, bt, wt, bt],
        out_specs=[blk, blk],
        out_shape=[jax.ShapeDtypeStruct((N, D), F32),
                   jax.ShapeDtypeStruct((N, D), F32)],
    )(h1, h2, c3, g3, wu1, wu2, bu, wt1, bt1, wt2, bt2)


def _tail1_body(c3_ref, g3_ref, wu1_ref, wu2_ref, bu_ref, wt1_ref, bt1_ref,
                wt2_ref, bt2_ref, o_ref):
    h3 = _h_from(c3_ref[...], g3_ref, wu1_ref, wu2_ref, bu_ref)
    mq, mc = _masked_transform(h3, wt1_ref, bt1_ref, wt2_ref, bt2_ref)
    plan3 = _plan_from(mq, mc)
    rows = []
    for p in range(PP):
        mqb = mq[MS * p:MS * (p + 1)]
        mcb = mc[MS * p:MS * (p + 1)]
        r = mqb - jnp.dot(plan3[p], mcb, preferred_element_type=F32)
        s = -jnp.sqrt(jnp.sum(r * r) + 1e-12)
        rows.append(jnp.full((1, 128), s, F32))
    o_ref[...] = jnp.concatenate(rows, axis=0)


def _tail1(c3, g3, wu1, wu2, bu, wt1, bt1, wt2, bt2):
    blk = pl.BlockSpec((PP * PAIR, D), lambda i: (i, 0))
    gblk = pl.BlockSpec((NC, PP * PAIR // 2, 2 * D), lambda i: (0, i, 0))
    wt = pl.BlockSpec((TD, TD), lambda i: (0, 0))
    bt = pl.BlockSpec((1, TD), lambda i: (0, 0))
    return pl.pallas_call(
        _tail1_body,
        grid=(TG,),
        in_specs=[blk, gblk, wt, wt, bt, wt, bt, wt, bt],
        out_specs=pl.BlockSpec((PP, 128), lambda i: (i, 0)),
        out_shape=jax.ShapeDtypeStruct((B, 128), F32),
    )(c3, g3, wu1, wu2, bu, wt1, bt1, wt2, bt2)


_SC_MESH = plsc.VectorSubcoreMesh(core_axis_name="c", subcore_axis_name="s")


@functools.partial(
    pl.kernel,
    out_type=jax.ShapeDtypeStruct((NC, N // 2, 2 * D), F32),
    mesh=_SC_MESH,
    compiler_params=pltpu.CompilerParams(use_tc_tiling_on_sc=False),
    scratch_types=[
        pltpu.VMEM((NCH, CH), jnp.int32),
        pltpu.VMEM((NCH, CH), jnp.int32),
        pltpu.VMEM((NCH, CH), jnp.int32),
        pltpu.VMEM((CH, D), F32),
        pltpu.VMEM((CH, D), F32),
        pltpu.VMEM((CH // 2, 2 * D), F32),
        pltpu.VMEM_SHARED((N, D), F32),
        pltpu.SemaphoreType.DMA,
        pltpu.SemaphoreType.DMA,
        pltpu.SemaphoreType.DMA,
    ],
)
def _edge_sc(ab_hbm, ce_hbm, f2_hbm, t2_hbm, t_hbm, out_hbm,
             fidx2, tidx2, tidx, buf_a, buf_b, buf_c, agg,
             sem_a, sem_b, sem_c):
    cid = lax.axis_index("c")
    sid = lax.axis_index("s")
    wid = cid * NS + sid

    def zrow(r, carry):
        for q in range(D // 16):
            buf_a[r, pl.ds(q * 16, 16)] = jnp.zeros((16,), F32)
        return carry

    lax.fori_loop(0, CH, zrow, 0)
    for k in range(ROWS_PER_TILE // CH):
        pltpu.sync_copy(buf_a, agg.at[pl.ds(sid * ROWS_PER_TILE + k * CH, CH)])
    plsc.subcore_barrier()

    pltpu.sync_copy(f2_hbm.at[wid], fidx2)
    pltpu.sync_copy(t2_hbm.at[wid], tidx2)
    pltpu.sync_copy(t_hbm.at[wid], tidx)

    def chunk(j, carry):
        ca = pltpu.async_copy(ab_hbm.at[fidx2.at[j]], buf_a, sem_a)
        cb = pltpu.async_copy(ab_hbm.at[tidx2.at[j]], buf_b, sem_b)
        cc = pltpu.async_copy(
            ce_hbm.at[pl.ds((wid * NCH + j) * (CH // 2), CH // 2)], buf_c,
            sem_c)
        ca.wait()
        cb.wait()
        cc.wait()

        def row(rp, inner):
            r0 = 2 * rp
            r1 = r0 + 1
            for q in range(D // 16):
                sl = pl.ds(q * 16, 16)
                v = buf_a[r0, sl] + buf_b[r0, sl] + buf_c[rp, sl]
                buf_a[r0, sl] = jnp.maximum(v, 0.0)
            for q in range(D // 16):
                sl = pl.ds(q * 16, 16)
                sl2 = pl.ds(D + q * 16, 16)
                v = buf_a[r1, sl] + buf_b[r1, sl] + buf_c[rp, sl2]
                buf_a[r1, sl] = jnp.maximum(v, 0.0)
            return inner

        lax.fori_loop(0, CH // 2, row, 0)
        pltpu.sync_copy(buf_a, agg.at[tidx.at[j]], add=True)
        return carry

    lax.fori_loop(0, NCH, chunk, 0)
    plsc.subcore_barrier()

    for k in range(ROWS_PER_TILE // CH):
        m = sid * (ROWS_PER_TILE // CH) + k
        pltpu.sync_copy(
            agg.at[pl.ds(m * CH, CH)],
            out_hbm.at[cid, pl.ds((m // 2) * CH, CH), pl.ds((m % 2) * D, D)])


def kernel(node_features, edge_features, from_idx, to_idx, W_enc_n, b_enc_n,
           W_enc_e, b_enc_e, Wc1, bc1, Wc2, bc2, W_msg, b_msg, W_upd, b_upd,
           Wt1, bt1, Wt2, bt2):
    order = jnp.asarray(_EDGE_ORDER)
    fi = from_idx.astype(jnp.int32)[order].reshape(NW, NCH, CH)
    ti = to_idx.astype(jnp.int32)[order].reshape(NW, NCH, CH)
    fi2 = fi * 2
    ti2 = ti * 2 + 1
    wc1a, wc1b = Wc1[:D], Wc1[D:]
    wma, wmb, wmc = W_msg[:D], W_msg[D:2 * D], W_msg[2 * D:]
    wu1, wu2 = W_upd[:D], W_upd[D:]
    bc1r = bc1.reshape(1, 2 * D)
    bc2r = bc2.reshape(1, D)
    bur = b_upd.reshape(1, D)
    bt1r = bt1.reshape(1, TD)
    bt2r = bt2.reshape(1, TD)

    ce2 = _enc_edges(edge_features, W_enc_e, b_enc_e.reshape(1, EENC), wmc,
                     b_msg.reshape(1, D))

    def edge(ab):
        g = _edge_sc(ab.reshape(2 * N, D), ce2, fi2, ti2, ti)
        return g, g

    c1, ab1 = _layer1(node_features, W_enc_n, b_enc_n.reshape(1, D),
                      wc1a, bc1r, Wc2, bc2r, wma, wmb)
    g1, gp1 = edge(ab1)
    h1, c2, ab2 = _layer_next(c1, gp1, wu1, wu2, bur, wc1a, bc1r, Wc2,
                              bc2r, wma, wmb)
    _, gp2 = edge(ab2)
    h2, c3, ab3 = _layer_next(c2, gp2, wu1, wu2, bur, wc1a, bc1r, Wc2,
                              bc2r, wma, wmb)
    _, gp3 = edge(ab3)
    s1, s2 = _tail0(h1, h2, c3, gp3, wu1, wu2, bur, Wt1, bt1r, Wt2, bt2r)
    _, c4, ab4 = _layer_next_wi(c1, gp1, s1, wu1, wu2, bur, wc1a, wc1b,
                                bc1r, Wc2, bc2r, wma, wmb)
    _, gp4 = edge(ab4)
    _, c5, ab5 = _layer_next_wi(c4, gp4, s2, wu1, wu2, bur, wc1a, wc1b,
                                bc1r, Wc2, bc2r, wma, wmb)
    _, gp5 = edge(ab5)
    out = _tail1(c5, gp5, wu1, wu2, bur, Wt1, bt1r, Wt2, bt2r)
    return out[:, 0]
